# baseline (speedup 1.0000x reference)
import jax
import jax.numpy as jnp
from jax.experimental import pallas as pl
from jax.experimental.pallas import tpu as pltpu

_V, _N, _Z = 8, 128, 2
_N_CHR, _BPC = 10, 8
_NB = _N_CHR * _BPC
_R = 320
_NT = 32
_NSTEPS = _N // _NT


def _block_mlp_kernel(x_ref, w320_ref, g0_ref, g1_ref, b1_ref, w2g_ref,
                      b2_ref, wa_ref, wc_ref, ba_ref, bc_ref,
                      actor_ref, critic_ref, cacc_ref):
    ni = pl.program_id(1)

    x = x_ref[0]
    p = x * w320_ref[...]
    u = jnp.sum(p, axis=-1)

    t0 = jnp.dot(u, g0_ref[...], preferred_element_type=jnp.float32)
    t1 = jnp.dot(u, g1_ref[...], preferred_element_type=jnp.float32)

    b1 = b1_ref[...]
    ye = jax.nn.relu(t0 + b1)
    yo = jax.nn.relu(t1 + b1)

    m = jnp.maximum(ye, yo)
    ee = jnp.exp(ye - m)
    eo = jnp.exp(yo - m)
    pooled = (ee * ye + eo * yo) / (ee + eo)

    feats = jax.nn.relu(
        jnp.dot(pooled, w2g_ref[...], preferred_element_type=jnp.float32)
        + b2_ref[...])

    av = jnp.sum(feats * wa_ref[...], axis=-1, keepdims=True)
    actor_ref[0] = av + ba_ref[0]

    cpart = jnp.sum(feats * wc_ref[...])

    @pl.when(ni == 0)
    def _():
        cacc_ref[0] = 0.0

    cacc_ref[0] += cpart

    @pl.when(ni == _NSTEPS - 1)
    def _():
        critic_ref[...] = jnp.full((1, 1, 128), cacc_ref[0] + bc_ref[0],
                                   dtype=jnp.float32)


@jax.jit
def _run(x, W1, b1, W2, b2, Wa, ba, Wc, bc):
    xp = (x.reshape(_V, _N, 160, 128, _Z)
          .transpose(0, 1, 2, 4, 3)
          .reshape(_V, _N, _R, 128))
    w320 = jnp.repeat(W1.reshape(_NB, 2, 128), 2, axis=1).reshape(_R, 128)
    r = jnp.arange(_R)[:, None]
    bcols = jnp.arange(_NB)[None, :]
    g0 = ((r // 4 == bcols) & (r % 2 == 0)).astype(jnp.float32)
    g1 = ((r // 4 == bcols) & (r % 2 == 1)).astype(jnp.float32)
    b1r = b1.reshape(1, _NB)
    w2flat = W2.reshape(_NB)
    w2g = jnp.where(
        jnp.arange(_NB)[:, None] // _BPC == jnp.arange(_N_CHR)[None, :],
        w2flat[:, None], 0.0).astype(jnp.float32)
    b2r = b2.reshape(1, _N_CHR)
    war = Wa.reshape(1, _N_CHR)
    wcr = (Wc / _N).reshape(1, _N_CHR)

    actor3, critic2 = pl.pallas_call(
        _block_mlp_kernel,
        grid=(_V, _NSTEPS),
        in_specs=[
            pl.BlockSpec((1, _NT, _R, 128), lambda v, i: (v, i, 0, 0)),
            pl.BlockSpec((_R, 128), lambda v, i: (0, 0)),
            pl.BlockSpec((_R, _NB), lambda v, i: (0, 0)),
            pl.BlockSpec((_R, _NB), lambda v, i: (0, 0)),
            pl.BlockSpec((1, _NB), lambda v, i: (0, 0)),
            pl.BlockSpec((_NB, _N_CHR), lambda v, i: (0, 0)),
            pl.BlockSpec((1, _N_CHR), lambda v, i: (0, 0)),
            pl.BlockSpec((1, _N_CHR), lambda v, i: (0, 0)),
            pl.BlockSpec((1, _N_CHR), lambda v, i: (0, 0)),
            pl.BlockSpec(memory_space=pltpu.SMEM),
            pl.BlockSpec(memory_space=pltpu.SMEM),
        ],
        out_specs=[
            pl.BlockSpec((1, _NT, 1), lambda v, i: (v, i, 0)),
            pl.BlockSpec((1, 1, 128), lambda v, i: (v, 0, 0)),
        ],
        out_shape=[
            jax.ShapeDtypeStruct((_V, _N, 1), jnp.float32),
            jax.ShapeDtypeStruct((_V, 1, 128), jnp.float32),
        ],
        scratch_shapes=[pltpu.SMEM((1,), jnp.float32)],
        compiler_params=pltpu.CompilerParams(
            dimension_semantics=("parallel", "arbitrary"),
            vmem_limit_bytes=50 * 1024 * 1024,
        ),
        name="block_mlp_fused",
    )(xp, w320, g0, g1, b1r, w2g, b2r, war, wcr, ba, bc)

    return actor3[..., 0], critic2[:, 0, 0]


def kernel(x, W1, b1, W2, b2, Wa, ba, Wc, bc):
    return _run(x, W1, b1, W2, b2, Wa, ba, Wc, bc)


# speedup vs baseline: 1.1519x; 1.1519x over previous
import jax
import jax.numpy as jnp
from jax.experimental import pallas as pl
from jax.experimental.pallas import tpu as pltpu

_V, _N, _Z = 8, 128, 2
_N_CHR, _BPC = 10, 8
_NB = _N_CHR * _BPC
_R = 320
_NT = 128
_NSTEPS = _N // _NT


def _block_mlp_kernel(x_ref, w320_ref, g0_ref, g1_ref, b1_ref, w2g_ref,
                      b2_ref, wa_ref, wc_ref, ba_ref, bc_ref,
                      actor_ref, critic_ref, cacc_ref):
    ni = pl.program_id(1)

    x = x_ref[0]
    p = x * w320_ref[...]
    u = jnp.sum(p, axis=-1)

    t0 = jnp.dot(u, g0_ref[...], preferred_element_type=jnp.float32)
    t1 = jnp.dot(u, g1_ref[...], preferred_element_type=jnp.float32)

    b1 = b1_ref[...]
    ye = jax.nn.relu(t0 + b1)
    yo = jax.nn.relu(t1 + b1)

    m = jnp.maximum(ye, yo)
    ee = jnp.exp(ye - m)
    eo = jnp.exp(yo - m)
    pooled = (ee * ye + eo * yo) / (ee + eo)

    feats = jax.nn.relu(
        jnp.dot(pooled, w2g_ref[...], preferred_element_type=jnp.float32)
        + b2_ref[...])

    av = jnp.sum(feats * wa_ref[...], axis=-1, keepdims=True)
    actor_ref[0] = av + ba_ref[0]

    cpart = jnp.sum(feats * wc_ref[...])

    @pl.when(ni == 0)
    def _():
        cacc_ref[0] = 0.0

    cacc_ref[0] += cpart

    @pl.when(ni == _NSTEPS - 1)
    def _():
        critic_ref[...] = jnp.full((1, 1, 128), cacc_ref[0] + bc_ref[0],
                                   dtype=jnp.float32)


@jax.jit
def _run(x, W1, b1, W2, b2, Wa, ba, Wc, bc):
    xp = (x.reshape(_V, _N, 160, 128, _Z)
          .transpose(0, 1, 2, 4, 3)
          .reshape(_V, _N, _R, 128))
    w320 = jnp.repeat(W1.reshape(_NB, 2, 128), 2, axis=1).reshape(_R, 128)
    r = jnp.arange(_R)[:, None]
    bcols = jnp.arange(_NB)[None, :]
    g0 = ((r // 4 == bcols) & (r % 2 == 0)).astype(jnp.float32)
    g1 = ((r // 4 == bcols) & (r % 2 == 1)).astype(jnp.float32)
    b1r = b1.reshape(1, _NB)
    w2flat = W2.reshape(_NB)
    w2g = jnp.where(
        jnp.arange(_NB)[:, None] // _BPC == jnp.arange(_N_CHR)[None, :],
        w2flat[:, None], 0.0).astype(jnp.float32)
    b2r = b2.reshape(1, _N_CHR)
    war = Wa.reshape(1, _N_CHR)
    wcr = (Wc / _N).reshape(1, _N_CHR)

    actor3, critic2 = pl.pallas_call(
        _block_mlp_kernel,
        grid=(_V, _NSTEPS),
        in_specs=[
            pl.BlockSpec((1, _NT, _R, 128), lambda v, i: (v, i, 0, 0)),
            pl.BlockSpec((_R, 128), lambda v, i: (0, 0)),
            pl.BlockSpec((_R, _NB), lambda v, i: (0, 0)),
            pl.BlockSpec((_R, _NB), lambda v, i: (0, 0)),
            pl.BlockSpec((1, _NB), lambda v, i: (0, 0)),
            pl.BlockSpec((_NB, _N_CHR), lambda v, i: (0, 0)),
            pl.BlockSpec((1, _N_CHR), lambda v, i: (0, 0)),
            pl.BlockSpec((1, _N_CHR), lambda v, i: (0, 0)),
            pl.BlockSpec((1, _N_CHR), lambda v, i: (0, 0)),
            pl.BlockSpec(memory_space=pltpu.SMEM),
            pl.BlockSpec(memory_space=pltpu.SMEM),
        ],
        out_specs=[
            pl.BlockSpec((1, _NT, 1), lambda v, i: (v, i, 0)),
            pl.BlockSpec((1, 1, 128), lambda v, i: (v, 0, 0)),
        ],
        out_shape=[
            jax.ShapeDtypeStruct((_V, _N, 1), jnp.float32),
            jax.ShapeDtypeStruct((_V, 1, 128), jnp.float32),
        ],
        scratch_shapes=[pltpu.SMEM((1,), jnp.float32)],
        compiler_params=pltpu.CompilerParams(
            dimension_semantics=("parallel", "arbitrary"),
            vmem_limit_bytes=60 * 1024 * 1024,
        ),
        name="block_mlp_fused",
    )(xp, w320, g0, g1, b1r, w2g, b2r, war, wcr, ba, bc)

    return actor3[..., 0], critic2[:, 0, 0]


def kernel(x, W1, b1, W2, b2, Wa, ba, Wc, bc):
    return _run(x, W1, b1, W2, b2, Wa, ba, Wc, bc)
